# Initial kernel scaffold; baseline (speedup 1.0000x reference)
import jax
import jax.numpy as jnp
from jax.experimental import pallas as pl


def kernel(x, neighbors_node, neighbors_relation, W, b):
    raise NotImplementedError("write your pallas kernel here")


# speedup vs baseline: 12.0344x; 12.0344x over previous
import functools

import jax
import jax.numpy as jnp
from jax import lax
from jax.experimental import pallas as pl
from jax.experimental.pallas import tpu as pltpu
from jax.experimental.pallas import tpu_sc as plsc

_NC = 2
_NS = 16
_NW = _NC * _NS
_LANES = 16

_GRP = 128
_NBUF = 4


def _sc_agg_body(ncaps, m, npw, groups, slots_w,
                 table, nbr, rel, out,
                 nbr_v, rel_v, src_v, dst_v, bufs, acc_sh,
                 sem0, sem1, sem2, sem3):
    gsems = (sem0, sem1, sem2, sem3)
    c = lax.axis_index("c")
    s = lax.axis_index("s")
    wid = c * _NS + s

    rows_w = npw * ncaps
    acc_base = s * slots_w

    pltpu.sync_copy(nbr.at[wid], nbr_v)
    pltpu.sync_copy(rel.at[wid], rel_v)
    pltpu.sync_copy(table.at[pl.ds(wid * rows_w, rows_w)],
                    acc_sh.at[pl.ds(acc_base, rows_w)])

    iota16 = lax.iota(jnp.int32, _LANES)
    ncaps16 = jnp.full((_LANES,), ncaps, jnp.int32)
    m16 = jnp.full((_LANES,), m, jnp.int32)
    base16 = jnp.full((_LANES,), acc_base, jnp.int32)

    def idx_row(r, carry):
        for t in range(_GRP // _LANES):
            sl = pl.ds(t * _LANES, _LANES)
            n16 = nbr_v[r, sl]
            r16 = rel_v[r, sl]
            off16 = jnp.full((_LANES,), r * _GRP + t * _LANES, jnp.int32)
            e16 = off16 + iota16
            src_v[r, sl] = n16 * ncaps16 + r16
            dst_v[r, sl] = base16 + lax.div(e16, m16) * ncaps16 + r16
        return carry

    lax.fori_loop(0, groups, idx_row, 0)

    def gather(g, b):
        return pltpu.async_copy(table.at[src_v.at[g]], bufs.at[b], gsems[b])

    def gather_wait(g, b):
        pltpu.make_async_copy(table.at[src_v.at[g]], bufs.at[b],
                              gsems[b]).wait()

    def scatter_add(g, b):
        pltpu.sync_copy(bufs.at[b], acc_sh.at[dst_v.at[g]], add=True)

    for b in range(_NBUF):
        gather(b, b)

    steady = groups // _NBUF - 1

    def grp_body(go, carry):
        for b in range(_NBUF):
            g = go * _NBUF + b
            gather_wait(g, b)
            scatter_add(g, b)
            gather(g + _NBUF, b)
        return carry

    lax.fori_loop(0, steady, grp_body, 0)

    for b in range(_NBUF):
        g = steady * _NBUF + b
        gather_wait(g, b)
        scatter_add(g, b)

    pltpu.sync_copy(acc_sh.at[pl.ds(acc_base, rows_w)],
                    out.at[pl.ds(wid * rows_w, rows_w)])


def _sc_aggregate(table, nbr, rel, ncaps, hidden, m, npw, groups, slots_w):
    mesh = plsc.VectorSubcoreMesh(core_axis_name="c", subcore_axis_name="s")
    out_rows = _NW * npw * ncaps
    body = functools.partial(_sc_agg_body, ncaps, m, npw, groups, slots_w)
    f = pl.kernel(
        body,
        out_type=jax.ShapeDtypeStruct((out_rows, hidden), jnp.float32),
        mesh=mesh,
        scratch_types=[
            pltpu.VMEM((groups, _GRP), jnp.int32),
            pltpu.VMEM((groups, _GRP), jnp.int32),
            pltpu.VMEM((groups, _GRP), jnp.int32),
            pltpu.VMEM((groups, _GRP), jnp.int32),
            pltpu.VMEM((_NBUF, _GRP, hidden), jnp.float32),
            pltpu.VMEM_SHARED((_NS * slots_w, hidden), jnp.float32),
            pltpu.SemaphoreType.DMA,
            pltpu.SemaphoreType.DMA,
            pltpu.SemaphoreType.DMA,
            pltpu.SemaphoreType.DMA,
        ],
        compiler_params=pltpu.CompilerParams(use_tc_tiling_on_sc=False),
    )
    return f(table, nbr, rel)


def _mm_body(xz_ref, w_ref, b_ref, o_ref):
    acc = lax.dot_general(xz_ref[...], w_ref[...],
                          (((1,), (1,)), ((), ())),
                          preferred_element_type=jnp.float32)
    o_ref[...] = jnp.maximum(acc + b_ref[...], 0.0)


def _dense_relu(xz, w, b2, n, d, blk):
    grid = n // blk
    return pl.pallas_call(
        _mm_body,
        grid=(grid,),
        in_specs=[
            pl.BlockSpec((blk, d), lambda i: (i, 0)),
            pl.BlockSpec((d, d), lambda i: (0, 0)),
            pl.BlockSpec((1, d), lambda i: (0, 0)),
        ],
        out_specs=pl.BlockSpec((blk, d), lambda i: (i, 0)),
        out_shape=jax.ShapeDtypeStruct((n, d), jnp.float32),
    )(xz, w, b2)


def kernel(x, neighbors_node, neighbors_relation, W, b):
    n, ncaps, hidden = x.shape
    m = neighbors_node.shape[0] // n
    d = ncaps * hidden

    npw = -(-n // _NW)
    n_pad = npw * _NW
    edges_w = npw * m
    groups = -(-edges_w // _GRP)
    groups += (-groups) % _NBUF
    edges_w_pad = groups * _GRP
    slots_w = (edges_w_pad // m) * ncaps

    nbr = neighbors_node.astype(jnp.int32)
    rel = neighbors_relation.astype(jnp.int32)

    table = jnp.pad(x.reshape(n * ncaps, hidden),
                    ((0, (n_pad - n) * ncaps + ncaps), (0, 0)))

    nbr_p = jnp.pad(nbr, (0, n_pad * m - n * m), constant_values=n)
    nbr_p = jnp.pad(nbr_p.reshape(_NW, edges_w),
                    ((0, 0), (0, edges_w_pad - edges_w)),
                    constant_values=n).reshape(_NW, groups, _GRP)
    rel_p = jnp.pad(rel, (0, n_pad * m - n * m))
    rel_p = jnp.pad(rel_p.reshape(_NW, edges_w),
                    ((0, 0), (0, edges_w_pad - edges_w))
                    ).reshape(_NW, groups, _GRP)

    xz = _sc_aggregate(table, nbr_p, rel_p,
                       ncaps, hidden, m, npw, groups, slots_w)
    xz = xz.reshape(n_pad, d)

    out = _dense_relu(xz, W, b.reshape(1, d), n, d, 400)
    return out.reshape(n, ncaps, hidden)


# speedup vs baseline: 14.0822x; 1.1702x over previous
import functools

import jax
import jax.numpy as jnp
from jax import lax
from jax.experimental import pallas as pl
from jax.experimental.pallas import tpu as pltpu
from jax.experimental.pallas import tpu_sc as plsc

_NC = 2
_NS = 16
_NW = _NC * _NS
_LANES = 16

_GRP = 128
_NBUF = 4


def _sc_agg_body(ncaps, m, npw, groups, slots_w, rows_last,
                 table, nbr, rel, out,
                 nbr_v, rel_v, src_v, dst_v, bufs, acc_sh,
                 sem0, sem1, sem2, sem3):
    gsems = (sem0, sem1, sem2, sem3)
    c = lax.axis_index("c")
    s = lax.axis_index("s")
    wid = c * _NS + s

    rows_w = npw * ncaps
    acc_base = s * slots_w

    pltpu.sync_copy(nbr.at[wid], nbr_v)
    pltpu.sync_copy(rel.at[wid], rel_v)

    @pl.when(wid < _NW - 1)
    def _():
        pltpu.sync_copy(table.at[pl.ds(wid * rows_w, rows_w)],
                        acc_sh.at[pl.ds(acc_base, rows_w)])

    @pl.when(wid == _NW - 1)
    def _():
        pltpu.sync_copy(table.at[pl.ds(wid * rows_w, rows_last)],
                        acc_sh.at[pl.ds(acc_base, rows_last)])

    iota16 = lax.iota(jnp.int32, _LANES)
    ncaps16 = jnp.full((_LANES,), ncaps, jnp.int32)
    m16 = jnp.full((_LANES,), m, jnp.int32)
    base16 = jnp.full((_LANES,), acc_base, jnp.int32)

    def idx_row(r, carry):
        for t in range(_GRP // _LANES):
            sl = pl.ds(t * _LANES, _LANES)
            n16 = nbr_v[r, sl]
            r16 = rel_v[r, sl]
            off16 = jnp.full((_LANES,), r * _GRP + t * _LANES, jnp.int32)
            e16 = off16 + iota16
            src_v[r, sl] = n16 * ncaps16 + r16
            dst_v[r, sl] = base16 + lax.div(e16, m16) * ncaps16 + r16
        return carry

    lax.fori_loop(0, groups, idx_row, 0)

    def gather(g, b):
        return pltpu.async_copy(table.at[src_v.at[g]], bufs.at[b], gsems[b])

    def gather_wait(g, b):
        pltpu.make_async_copy(table.at[src_v.at[g]], bufs.at[b],
                              gsems[b]).wait()

    def scatter_add(g, b):
        pltpu.sync_copy(bufs.at[b], acc_sh.at[dst_v.at[g]], add=True)

    for b in range(_NBUF):
        gather(b, b)

    steady = groups // _NBUF - 1

    def grp_body(go, carry):
        for b in range(_NBUF):
            g = go * _NBUF + b
            gather_wait(g, b)
            scatter_add(g, b)
            gather(g + _NBUF, b)
        return carry

    lax.fori_loop(0, steady, grp_body, 0)

    for b in range(_NBUF):
        g = steady * _NBUF + b
        gather_wait(g, b)
        scatter_add(g, b)

    pltpu.sync_copy(acc_sh.at[pl.ds(acc_base, rows_w)],
                    out.at[pl.ds(wid * rows_w, rows_w)])


def _sc_aggregate(table, nbr, rel, ncaps, hidden, m, npw, groups, slots_w,
                  rows_last):
    mesh = plsc.VectorSubcoreMesh(core_axis_name="c", subcore_axis_name="s")
    out_rows = _NW * npw * ncaps
    body = functools.partial(_sc_agg_body, ncaps, m, npw, groups, slots_w,
                             rows_last)
    f = pl.kernel(
        body,
        out_type=jax.ShapeDtypeStruct((out_rows, hidden), jnp.float32),
        mesh=mesh,
        scratch_types=[
            pltpu.VMEM((groups, _GRP), jnp.int32),
            pltpu.VMEM((groups, _GRP), jnp.int32),
            pltpu.VMEM((groups, _GRP), jnp.int32),
            pltpu.VMEM((groups, _GRP), jnp.int32),
            pltpu.VMEM((_NBUF, _GRP, hidden), jnp.float32),
            pltpu.VMEM_SHARED((_NS * slots_w, hidden), jnp.float32),
            pltpu.SemaphoreType.DMA,
            pltpu.SemaphoreType.DMA,
            pltpu.SemaphoreType.DMA,
            pltpu.SemaphoreType.DMA,
        ],
        compiler_params=pltpu.CompilerParams(use_tc_tiling_on_sc=False),
    )
    return f(table, nbr, rel)


def _mm_body(xz_ref, w_ref, b_ref, o_ref):
    acc = lax.dot_general(xz_ref[...], w_ref[...],
                          (((1,), (1,)), ((), ())),
                          preferred_element_type=jnp.float32)
    o_ref[...] = jnp.maximum(acc + b_ref[...], 0.0)


def _dense_relu(xz, w, b2, n, d, blk):
    grid = n // blk
    return pl.pallas_call(
        _mm_body,
        grid=(grid,),
        in_specs=[
            pl.BlockSpec((blk, d), lambda i: (i, 0)),
            pl.BlockSpec((d, d), lambda i: (0, 0)),
            pl.BlockSpec((1, d), lambda i: (0, 0)),
        ],
        out_specs=pl.BlockSpec((blk, d), lambda i: (i, 0)),
        out_shape=jax.ShapeDtypeStruct((n, d), jnp.float32),
    )(xz, w, b2)


def kernel(x, neighbors_node, neighbors_relation, W, b):
    n, ncaps, hidden = x.shape
    m = neighbors_node.shape[0] // n
    d = ncaps * hidden

    npw = -(-n // _NW)
    n_pad = npw * _NW
    edges_w = npw * m
    groups = -(-edges_w // _GRP)
    groups += (-groups) % _NBUF
    edges_w_pad = groups * _GRP
    slots_w = (edges_w_pad // m) * ncaps

    nbr = neighbors_node.astype(jnp.int32)
    rel = neighbors_relation.astype(jnp.int32)

    table = x.reshape(n * ncaps, hidden)

    nbr_p = jnp.pad(nbr, (0, n_pad * m - n * m))
    nbr_p = jnp.pad(nbr_p.reshape(_NW, edges_w),
                    ((0, 0), (0, edges_w_pad - edges_w))
                    ).reshape(_NW, groups, _GRP)
    rel_p = jnp.pad(rel, (0, n_pad * m - n * m))
    rel_p = jnp.pad(rel_p.reshape(_NW, edges_w),
                    ((0, 0), (0, edges_w_pad - edges_w))
                    ).reshape(_NW, groups, _GRP)

    rows_last = (n - (_NW - 1) * npw) * ncaps
    xz = _sc_aggregate(table, nbr_p, rel_p,
                       ncaps, hidden, m, npw, groups, slots_w, rows_last)
    xz = xz.reshape(n_pad, d)

    out = _dense_relu(xz, W, b.reshape(1, d), n, d, 400)
    return out.reshape(n, ncaps, hidden)


# speedup vs baseline: 20.5479x; 1.4591x over previous
import functools

import jax
import jax.numpy as jnp
from jax import lax
from jax.experimental import pallas as pl
from jax.experimental.pallas import tpu as pltpu
from jax.experimental.pallas import tpu_sc as plsc

_NC = 2
_NS = 16
_NW = _NC * _NS
_LANES = 16

_GRP = 128
_NBUF = 16
_NWB = 2


def _sc_agg_body(ncaps, m, npw, groups, slots_w, rows_last,
                 table, srcg, dstg, out,
                 src_v, dst_v, bufs, acc_sh,
                 sem_init, *sems):
    wbsems = sems[:_NWB]
    gsems = sems[_NWB:]
    c = lax.axis_index("c")
    s = lax.axis_index("s")
    wid = c * _NS + s

    rows_w = npw * ncaps
    acc_base = s * slots_w

    def init_copy(rows):
        return pltpu.make_async_copy(table.at[pl.ds(wid * rows_w, rows)],
                                     acc_sh.at[pl.ds(acc_base, rows)],
                                     sem_init)

    @pl.when(wid < _NW - 1)
    def _():
        init_copy(rows_w).start()

    @pl.when(wid == _NW - 1)
    def _():
        init_copy(rows_last).start()

    pltpu.sync_copy(srcg.at[wid], src_v)
    pltpu.sync_copy(dstg.at[wid], dst_v)

    def gather(g, b):
        return pltpu.async_copy(table.at[src_v.at[g]], bufs.at[b], gsems[b])

    def gather_wait(g, b):
        pltpu.make_async_copy(table.at[src_v.at[g]], bufs.at[b],
                              gsems[b]).wait()

    def scatter_add(g, b):
        pltpu.sync_copy(bufs.at[b], acc_sh.at[dst_v.at[g]], add=True)

    for b in range(_NBUF):
        gather(b, b)

    @pl.when(wid < _NW - 1)
    def _():
        init_copy(rows_w).wait()

    @pl.when(wid == _NW - 1)
    def _():
        init_copy(rows_last).wait()

    steady = groups // _NBUF - 1
    chunk = _NBUF * (_GRP // m) * ncaps

    def wb_copy(go, rows):
        return pltpu.make_async_copy(
            acc_sh.at[pl.ds(acc_base + go * chunk, rows)],
            out.at[pl.ds(wid * rows_w + go * chunk, rows)],
            wbsems[go % _NWB])

    for go in range(steady):
        for b in range(_NBUF):
            g = go * _NBUF + b
            gather_wait(g, b)
            scatter_add(g, b)
            gather(g + _NBUF, b)
        if go >= _NWB:
            wb_copy(go - _NWB, chunk).wait()
        wb_copy(go, chunk).start()

    for b in range(_NBUF):
        g = steady * _NBUF + b
        gather_wait(g, b)
        scatter_add(g, b)

    for go in range(steady - _NWB, steady):
        wb_copy(go, chunk).wait()
    tail = rows_w - steady * chunk
    pltpu.sync_copy(acc_sh.at[pl.ds(acc_base + steady * chunk, tail)],
                    out.at[pl.ds(wid * rows_w + steady * chunk, tail)])


def _sc_aggregate(table, srcg, dstg, ncaps, hidden, m, npw, groups, slots_w,
                  rows_last):
    mesh = plsc.VectorSubcoreMesh(core_axis_name="c", subcore_axis_name="s")
    out_rows = _NW * npw * ncaps
    body = functools.partial(_sc_agg_body, ncaps, m, npw, groups, slots_w,
                             rows_last)
    f = pl.kernel(
        body,
        out_type=jax.ShapeDtypeStruct((out_rows, hidden), jnp.float32),
        mesh=mesh,
        scratch_types=[
            pltpu.VMEM((groups, _GRP), jnp.int32),
            pltpu.VMEM((groups, _GRP), jnp.int32),
            pltpu.VMEM((_NBUF, _GRP, hidden), jnp.float32),
            pltpu.VMEM_SHARED((_NS * slots_w, hidden), jnp.float32),
        ] + [pltpu.SemaphoreType.DMA] * (1 + _NWB + _NBUF),
        compiler_params=pltpu.CompilerParams(use_tc_tiling_on_sc=False),
    )
    return f(table, srcg, dstg)


def _tr_body(xt_ref, o_ref):
    o_ref[...] = xt_ref[...].T


def _to_table(x, n, d):
    xt = jnp.transpose(x, (1, 2, 0)).reshape(d, n)
    return pl.pallas_call(
        _tr_body,
        in_specs=[pl.BlockSpec((d, n), lambda: (0, 0))],
        out_specs=pl.BlockSpec((n, d), lambda: (0, 0)),
        out_shape=jax.ShapeDtypeStruct((n, d), jnp.float32),
    )(xt)


def _mm_body(xz_ref, w_ref, b_ref, o_ref):
    acc = lax.dot_general(xz_ref[...], w_ref[...],
                          (((1,), (1,)), ((), ())),
                          preferred_element_type=jnp.float32)
    o_ref[...] = jnp.maximum(acc + b_ref[...], 0.0)


def _dense_relu(xz, w, b2, n, d, blk):
    grid = n // blk
    return pl.pallas_call(
        _mm_body,
        grid=(grid,),
        in_specs=[
            pl.BlockSpec((blk, d), lambda i: (i, 0)),
            pl.BlockSpec((d, d), lambda i: (0, 0)),
            pl.BlockSpec((1, d), lambda i: (0, 0)),
        ],
        out_specs=pl.BlockSpec((blk, d), lambda i: (i, 0)),
        out_shape=jax.ShapeDtypeStruct((n, d), jnp.float32),
    )(xz, w, b2)


def kernel(x, neighbors_node, neighbors_relation, W, b):
    n, ncaps, hidden = x.shape
    m = neighbors_node.shape[0] // n
    d = ncaps * hidden

    npw = -(-n // _NW)
    n_pad = npw * _NW
    edges_w = npw * m
    groups = -(-edges_w // _GRP)
    groups += (-groups) % _NBUF
    edges_w_pad = groups * _GRP
    slots_w = (edges_w_pad // m) * ncaps

    nbr = neighbors_node.astype(jnp.int32)
    rel = neighbors_relation.astype(jnp.int32)

    table = _to_table(x, n, d).reshape(n * ncaps, hidden)

    nbr_p = jnp.pad(nbr, (0, n_pad * m - n * m))
    nbr_p = jnp.pad(nbr_p.reshape(_NW, edges_w),
                    ((0, 0), (0, edges_w_pad - edges_w)))
    rel_p = jnp.pad(rel, (0, n_pad * m - n * m))
    rel_p = jnp.pad(rel_p.reshape(_NW, edges_w),
                    ((0, 0), (0, edges_w_pad - edges_w)))
    src_p = (nbr_p * ncaps + rel_p).reshape(_NW, groups, _GRP)
    e_loc = jnp.arange(edges_w_pad, dtype=jnp.int32)
    s_of_w = (jnp.arange(_NW, dtype=jnp.int32) % _NS)[:, None]
    dst_p = (s_of_w * slots_w + (e_loc // m)[None, :] * ncaps + rel_p
             ).reshape(_NW, groups, _GRP)

    rows_last = (n - (_NW - 1) * npw) * ncaps
    xz = _sc_aggregate(table, src_p, dst_p,
                       ncaps, hidden, m, npw, groups, slots_w, rows_last)
    xz = xz.reshape(n_pad, d)

    out = _dense_relu(xz, W, b.reshape(1, d), n, d, 400)
    return out.reshape(n, ncaps, hidden)
